# baseline (speedup 1.0000x reference)
import jax
import jax.numpy as jnp
from jax import lax
from jax.experimental import pallas as pl

_COUNT = 10
_BUCKET_SIZE = 3


def _fused_body(x_ref, ply_ref, w_ref, b_ref, o_ref):
    full_t = lax.dot_general(
        w_ref[...], x_ref[...],
        dimension_numbers=(((1,), (1,)), ((), ())),
        preferred_element_type=jnp.float32,
        precision=lax.Precision.DEFAULT,
    )
    c = ply_ref[0] // _BUCKET_SIZE
    rows = lax.broadcasted_iota(jnp.int32, (_COUNT, 1), 0)
    mask = c == rows
    sel = jnp.sum(jnp.where(mask, full_t + b_ref[...], 0.0), axis=0, keepdims=True)
    o_ref[0] = sel


def kernel(x, ply, W, b):
    batch, d = x.shape
    blk = 1024
    nblk = batch // blk
    out = pl.pallas_call(
        _fused_body,
        grid=(nblk,),
        in_specs=[
            pl.BlockSpec((blk, d), lambda i: (i, 0)),
            pl.BlockSpec((1, 1, blk), lambda i: (i, 0, 0)),
            pl.BlockSpec((_COUNT, d), lambda i: (0, 0)),
            pl.BlockSpec((_COUNT, 1), lambda i: (0, 0)),
        ],
        out_specs=pl.BlockSpec((1, 1, blk), lambda i: (i, 0, 0)),
        out_shape=jax.ShapeDtypeStruct((nblk, 1, blk), jnp.float32),
    )(x, ply.reshape(nblk, 1, blk), W, b.reshape(_COUNT, 1))
    return out.reshape(batch, 1)


# speedup vs baseline: 1.9340x; 1.9340x over previous
import jax
import jax.numpy as jnp
from jax import lax
from jax.experimental import pallas as pl

_COUNT = 10
_BUCKET_SIZE = 3


def _fused_body(x_ref, ply_ref, w_ref, b_ref, o_ref):
    full_t = lax.dot_general(
        w_ref[...], x_ref[...],
        dimension_numbers=(((1,), (1,)), ((), ())),
        preferred_element_type=jnp.float32,
        precision=lax.Precision.DEFAULT,
    )
    c = ply_ref[0] // _BUCKET_SIZE
    rows = lax.broadcasted_iota(jnp.int32, (_COUNT, 1), 0)
    mask = c == rows
    sel = jnp.sum(jnp.where(mask, full_t + b_ref[...], 0.0), axis=0, keepdims=True)
    o_ref[0] = sel


def kernel(x, ply, W, b):
    batch, d = x.shape
    blk = 4096
    nblk = batch // blk
    out = pl.pallas_call(
        _fused_body,
        grid=(nblk,),
        in_specs=[
            pl.BlockSpec((blk, d), lambda i: (i, 0)),
            pl.BlockSpec((1, 1, blk), lambda i: (i, 0, 0)),
            pl.BlockSpec((_COUNT, d), lambda i: (0, 0)),
            pl.BlockSpec((_COUNT, 1), lambda i: (0, 0)),
        ],
        out_specs=pl.BlockSpec((1, 1, blk), lambda i: (i, 0, 0)),
        out_shape=jax.ShapeDtypeStruct((nblk, 1, blk), jnp.float32),
    )(x, ply.reshape(nblk, 1, blk), W, b.reshape(_COUNT, 1))
    return out.reshape(batch, 1)


# speedup vs baseline: 2.1366x; 1.1047x over previous
import jax
import jax.numpy as jnp
from jax import lax
from jax.experimental import pallas as pl

_COUNT = 10
_BUCKET_SIZE = 3


def _fused_body(x_ref, ply_ref, w_ref, b_ref, o_ref):
    full_t = lax.dot_general(
        w_ref[...], x_ref[...],
        dimension_numbers=(((1,), (1,)), ((), ())),
        preferred_element_type=jnp.float32,
        precision=lax.Precision.DEFAULT,
    )
    c = ply_ref[0] // _BUCKET_SIZE
    rows = lax.broadcasted_iota(jnp.int32, (_COUNT, 1), 0)
    mask = c == rows
    sel = jnp.sum(jnp.where(mask, full_t + b_ref[...], 0.0), axis=0, keepdims=True)
    o_ref[0] = sel


def kernel(x, ply, W, b):
    batch, d = x.shape
    blk = 8192
    nblk = batch // blk
    out = pl.pallas_call(
        _fused_body,
        grid=(nblk,),
        in_specs=[
            pl.BlockSpec((blk, d), lambda i: (i, 0)),
            pl.BlockSpec((1, 1, blk), lambda i: (i, 0, 0)),
            pl.BlockSpec((_COUNT, d), lambda i: (0, 0)),
            pl.BlockSpec((_COUNT, 1), lambda i: (0, 0)),
        ],
        out_specs=pl.BlockSpec((1, 1, blk), lambda i: (i, 0, 0)),
        out_shape=jax.ShapeDtypeStruct((nblk, 1, blk), jnp.float32),
    )(x, ply.reshape(nblk, 1, blk), W, b.reshape(_COUNT, 1))
    return out.reshape(batch, 1)
